# Initial kernel scaffold; baseline (speedup 1.0000x reference)
import jax
import jax.numpy as jnp
from jax.experimental import pallas as pl


def kernel(x, edge_index, batch, link_indices, W1, b1, g1, be1, W2, b2, R2w, R2b, g2, be2, W3, b3, R3w, R3b, g3, be3, W4, b4, R4w, R4b, g4, be4, M1w, M1b, M2w, M2b, M3w, M3b):
    raise NotImplementedError("write your pallas kernel here")


# speedup vs baseline: 2.4018x; 2.4018x over previous
import functools

import jax
import jax.numpy as jnp
from jax.experimental import pallas as pl

N = 10000
D = 128
H = 128
H2 = 64
H3 = 32
OUT = 64
G = 1024
L = 4096

BN = 1000
NB = N // BN
BL = 1024
NLB = L // BL

_f32 = jnp.float32


def _dot(a, b):
    return jax.lax.dot_general(a, b, (((1,), (0,)), ((), ())),
                               preferred_element_type=_f32)


def _ka_body(x_ref, w1_ref, degp_ref, h1_ref, h1s_ref, dinv_ref, dinv2_ref):
    deg = degp_ref[0] + degp_ref[1] + 1.0
    dinv = jax.lax.rsqrt(deg)
    h1 = _dot(x_ref[...], w1_ref[...])
    h1_ref[...] = h1
    h1s_ref[...] = h1 * dinv
    dinv_ref[...] = dinv
    dinv2_ref[...] = dinv * dinv


def _stage_a(x, w1, degp):
    return pl.pallas_call(
        _ka_body,
        grid=(NB,),
        in_specs=[
            pl.BlockSpec((BN, D), lambda i: (i, 0)),
            pl.BlockSpec((D, H), lambda i: (0, 0)),
            pl.BlockSpec((2, BN, 1), lambda i: (0, i, 0)),
        ],
        out_specs=[
            pl.BlockSpec((BN, H), lambda i: (i, 0)),
            pl.BlockSpec((BN, H), lambda i: (i, 0)),
            pl.BlockSpec((BN, 1), lambda i: (i, 0)),
            pl.BlockSpec((BN, 1), lambda i: (i, 0)),
        ],
        out_shape=[
            jax.ShapeDtypeStruct((N, H), _f32),
            jax.ShapeDtypeStruct((N, H), _f32),
            jax.ShapeDtypeStruct((N, 1), _f32),
            jax.ShapeDtypeStruct((N, 1), _f32),
        ],
    )(x, w1, degp)


def _kb_body(sp_ref, h_ref, res_ref, dinv_ref, dinv2_ref, b_ref, g_ref,
             be_ref, wn_ref, rw_ref, rb_ref, hn_ref, rn_ref, hsn_ref):
    dinv = dinv_ref[...]
    t = (dinv * (sp_ref[0] + sp_ref[1]) + h_ref[...] * dinv2_ref[...]
         + b_ref[...] + res_ref[...])
    m = jnp.mean(t, axis=1, keepdims=True)
    c = t - m
    v = jnp.mean(c * c, axis=1, keepdims=True)
    y = jnp.maximum(c * jax.lax.rsqrt(v + 1e-5) * g_ref[...] + be_ref[...],
                    0.0)
    hn = _dot(y, wn_ref[...])
    hn_ref[...] = hn
    rn_ref[...] = _dot(y, rw_ref[...]) + rb_ref[...]
    hsn_ref[...] = hn * dinv


def _stage_b(sp, h, res, dinv, dinv2, b, g, be, wn, rw, rb):
    fi = h.shape[1]
    fn = wn.shape[1]
    return pl.pallas_call(
        _kb_body,
        grid=(NB,),
        in_specs=[
            pl.BlockSpec((2, BN, fi), lambda i: (0, i, 0)),
            pl.BlockSpec((BN, fi), lambda i: (i, 0)),
            pl.BlockSpec((BN, fi), lambda i: (i, 0)),
            pl.BlockSpec((BN, 1), lambda i: (i, 0)),
            pl.BlockSpec((BN, 1), lambda i: (i, 0)),
            pl.BlockSpec((1, fi), lambda i: (0, 0)),
            pl.BlockSpec((1, fi), lambda i: (0, 0)),
            pl.BlockSpec((1, fi), lambda i: (0, 0)),
            pl.BlockSpec((fi, fn), lambda i: (0, 0)),
            pl.BlockSpec((fi, fn), lambda i: (0, 0)),
            pl.BlockSpec((1, fn), lambda i: (0, 0)),
        ],
        out_specs=[
            pl.BlockSpec((BN, fn), lambda i: (i, 0)),
            pl.BlockSpec((BN, fn), lambda i: (i, 0)),
            pl.BlockSpec((BN, fn), lambda i: (i, 0)),
        ],
        out_shape=[
            jax.ShapeDtypeStruct((N, fn), _f32),
            jax.ShapeDtypeStruct((N, fn), _f32),
            jax.ShapeDtypeStruct((N, fn), _f32),
        ],
    )(sp, h, res, dinv, dinv2, b.reshape(1, fi), g.reshape(1, fi),
      be.reshape(1, fi), wn, rw, rb.reshape(1, fn))


def _kl_body(sp_ref, h_ref, res_ref, dinv_ref, dinv2_ref, b_ref, g_ref,
             be_ref, x4_ref):
    t = (dinv_ref[...] * (sp_ref[0] + sp_ref[1])
         + h_ref[...] * dinv2_ref[...] + b_ref[...] + res_ref[...])
    m = jnp.mean(t, axis=1, keepdims=True)
    c = t - m
    v = jnp.mean(c * c, axis=1, keepdims=True)
    x4_ref[...] = jnp.maximum(
        c * jax.lax.rsqrt(v + 1e-5) * g_ref[...] + be_ref[...], 0.0)


def _stage_last(sp, h, res, dinv, dinv2, b, g, be):
    fi = h.shape[1]
    return pl.pallas_call(
        _kl_body,
        grid=(NB,),
        in_specs=[
            pl.BlockSpec((2, BN, fi), lambda i: (0, i, 0)),
            pl.BlockSpec((BN, fi), lambda i: (i, 0)),
            pl.BlockSpec((BN, fi), lambda i: (i, 0)),
            pl.BlockSpec((BN, 1), lambda i: (i, 0)),
            pl.BlockSpec((BN, 1), lambda i: (i, 0)),
            pl.BlockSpec((1, fi), lambda i: (0, 0)),
            pl.BlockSpec((1, fi), lambda i: (0, 0)),
            pl.BlockSpec((1, fi), lambda i: (0, 0)),
        ],
        out_specs=pl.BlockSpec((BN, fi), lambda i: (i, 0)),
        out_shape=jax.ShapeDtypeStruct((N, fi), _f32),
    )(sp, h, res, dinv, dinv2, b.reshape(1, fi), g.reshape(1, fi),
      be.reshape(1, fi))


def _kc_body(x4_ref, batch_ref, sums_ref, cnt_ref):
    i = pl.program_id(0)
    seg = jax.lax.broadcasted_iota(jnp.int32, (G, 1), 0)
    oh = jnp.where(batch_ref[0] == seg, 1.0, 0.0)
    psum = _dot(oh, x4_ref[...])
    pcnt = jnp.sum(oh, axis=1, keepdims=True)

    @pl.when(i == 0)
    def _():
        sums_ref[...] = psum
        cnt_ref[...] = pcnt

    @pl.when(i != 0)
    def _():
        sums_ref[...] += psum
        cnt_ref[...] += pcnt


def _stage_pool(x4, batch3):
    return pl.pallas_call(
        _kc_body,
        grid=(NB,),
        in_specs=[
            pl.BlockSpec((BN, OUT), lambda i: (i, 0)),
            pl.BlockSpec((1, 1, BN), lambda i: (i, 0, 0)),
        ],
        out_specs=[
            pl.BlockSpec((G, OUT), lambda i: (0, 0)),
            pl.BlockSpec((G, 1), lambda i: (0, 0)),
        ],
        out_shape=[
            jax.ShapeDtypeStruct((G, OUT), _f32),
            jax.ShapeDtypeStruct((G, 1), _f32),
        ],
    )(x4, batch3)


def _kd_body(sums_ref, cnt_ref, li0_ref, li1_ref, m1a_ref, m1b_ref,
             m1bias_ref, m2w_ref, m2b_ref, m3w_ref, m3b_ref, out_ref):
    emb = sums_ref[...] / jnp.maximum(cnt_ref[...], 1.0)
    ea = _dot(emb, m1a_ref[...])
    eb = _dot(emb, m1b_ref[...])
    gid = jax.lax.broadcasted_iota(jnp.int32, (1, G), 1)
    oh0 = jnp.where(li0_ref[0].reshape(BL, 1) == gid, 1.0, 0.0)
    oh1 = jnp.where(li1_ref[0].reshape(BL, 1) == gid, 1.0, 0.0)
    h = jnp.maximum(_dot(oh0, ea) + _dot(oh1, eb) + m1bias_ref[...], 0.0)
    h = jnp.maximum(_dot(h, m2w_ref[...]) + m2b_ref[...], 0.0)
    out_ref[...] = jax.nn.sigmoid(_dot(h, m3w_ref[...]) + m3b_ref[...])


def _stage_links(sums, cnt, li0_3, li1_3, m1w, m1b, m2w, m2b, m3w, m3b):
    return pl.pallas_call(
        _kd_body,
        grid=(NLB,),
        in_specs=[
            pl.BlockSpec((G, OUT), lambda i: (0, 0)),
            pl.BlockSpec((G, 1), lambda i: (0, 0)),
            pl.BlockSpec((1, 1, BL), lambda i: (i, 0, 0)),
            pl.BlockSpec((1, 1, BL), lambda i: (i, 0, 0)),
            pl.BlockSpec((OUT, OUT), lambda i: (0, 0)),
            pl.BlockSpec((OUT, OUT), lambda i: (0, 0)),
            pl.BlockSpec((1, OUT), lambda i: (0, 0)),
            pl.BlockSpec((OUT, OUT // 2), lambda i: (0, 0)),
            pl.BlockSpec((1, OUT // 2), lambda i: (0, 0)),
            pl.BlockSpec((OUT // 2, 1), lambda i: (0, 0)),
            pl.BlockSpec((1, 1), lambda i: (0, 0)),
        ],
        out_specs=pl.BlockSpec((BL, 1), lambda i: (i, 0)),
        out_shape=jax.ShapeDtypeStruct((L, 1), _f32),
    )(sums, cnt, li0_3, li1_3, m1w[:OUT], m1w[OUT:], m1b.reshape(1, OUT),
      m2w, m2b.reshape(1, OUT // 2), m3w, m3b.reshape(1, 1))


def _scatter_parts(hs, src, dst):
    half = src.shape[0] // 2
    p0 = jnp.zeros_like(hs).at[dst[:half]].add(hs[src[:half]],
                                               mode='drop')
    p1 = jnp.zeros_like(hs).at[dst[half:]].add(hs[src[half:]],
                                               mode='drop')
    return jnp.stack([p0, p1])


def kernel(x, edge_index, batch, link_indices, W1, b1, g1, be1, W2, b2, R2w,
           R2b, g2, be2, W3, b3, R3w, R3b, g3, be3, W4, b4, R4w, R4b, g4,
           be4, M1w, M1b, M2w, M2b, M3w, M3b):
    src = edge_index[0]
    dst = edge_index[1]

    degp0 = jnp.zeros((N,), _f32).at[dst[:src.shape[0] // 2]].add(1.0)
    degp1 = jnp.zeros((N,), _f32).at[dst[src.shape[0] // 2:]].add(1.0)
    degp = jnp.stack([degp0, degp1]).reshape(2, N, 1)

    h1, h1s, dinv, dinv2 = _stage_a(x, W1, degp)

    s1 = _scatter_parts(h1s, src, dst)
    h2, r2, h2s = _stage_b(s1, h1, x, dinv, dinv2, b1, g1, be1, W2, R2w, R2b)

    s2 = _scatter_parts(h2s, src, dst)
    h3, r3, h3s = _stage_b(s2, h2, r2, dinv, dinv2, b2, g2, be2, W3, R3w,
                           R3b)

    s3 = _scatter_parts(h3s, src, dst)
    h4, r4, h4s = _stage_b(s3, h3, r3, dinv, dinv2, b3, g3, be3, W4, R4w,
                           R4b)

    s4 = _scatter_parts(h4s, src, dst)
    x4 = _stage_last(s4, h4, r4, dinv, dinv2, b4, g4, be4)

    batch3 = batch.reshape(NB, 1, BN)
    sums, cnt = _stage_pool(x4, batch3)

    li0_3 = link_indices[0].reshape(NLB, 1, BL)
    li1_3 = link_indices[1].reshape(NLB, 1, BL)
    out = _stage_links(sums, cnt, li0_3, li1_3, M1w, M1b, M2w, M2b, M3w,
                       M3b)
    return out.reshape(L)


# speedup vs baseline: 8.0113x; 3.3356x over previous
import functools

import jax
import jax.numpy as jnp
from jax import lax
from jax.experimental import pallas as pl
from jax.experimental.pallas import tpu as pltpu
from jax.experimental.pallas import tpu_sc as plsc

N = 10000
D = 128
H = 128
H2 = 64
H3 = 32
OUT = 64
G = 1024
L = 4096

BN = 1000
NB = N // BN
BL = 1024
NLB = L // BL

_f32 = jnp.float32


def _dot(a, b):
    return jax.lax.dot_general(a, b, (((1,), (0,)), ((), ())),
                               preferred_element_type=_f32)


def _ka_body(x_ref, w1_ref, degp_ref, h1_ref, h1s_ref, dinv_ref, dinv2_ref):
    deg = degp_ref[0] + degp_ref[1] + 1.0
    dinv = jax.lax.rsqrt(deg)
    h1 = _dot(x_ref[...], w1_ref[...])
    h1_ref[...] = h1
    h1s_ref[...] = h1 * dinv
    dinv_ref[...] = dinv
    dinv2_ref[...] = dinv * dinv


def _stage_a(x, w1, degp):
    return pl.pallas_call(
        _ka_body,
        grid=(NB,),
        in_specs=[
            pl.BlockSpec((BN, D), lambda i: (i, 0)),
            pl.BlockSpec((D, H), lambda i: (0, 0)),
            pl.BlockSpec((2, BN, 1), lambda i: (0, i, 0)),
        ],
        out_specs=[
            pl.BlockSpec((BN, H), lambda i: (i, 0)),
            pl.BlockSpec((BN, H), lambda i: (i, 0)),
            pl.BlockSpec((BN, 1), lambda i: (i, 0)),
            pl.BlockSpec((BN, 1), lambda i: (i, 0)),
        ],
        out_shape=[
            jax.ShapeDtypeStruct((N, H), _f32),
            jax.ShapeDtypeStruct((N, H), _f32),
            jax.ShapeDtypeStruct((N, 1), _f32),
            jax.ShapeDtypeStruct((N, 1), _f32),
        ],
    )(x, w1, degp)


def _kb_body(sp_ref, h_ref, res_ref, dinv_ref, dinv2_ref, b_ref, g_ref,
             be_ref, wn_ref, rw_ref, rb_ref, hn_ref, rn_ref, hsn_ref):
    dinv = dinv_ref[...]
    t = (dinv * (sp_ref[0] + sp_ref[1]) + h_ref[...] * dinv2_ref[...]
         + b_ref[...] + res_ref[...])
    m = jnp.mean(t, axis=1, keepdims=True)
    c = t - m
    v = jnp.mean(c * c, axis=1, keepdims=True)
    y = jnp.maximum(c * jax.lax.rsqrt(v + 1e-5) * g_ref[...] + be_ref[...],
                    0.0)
    hn = _dot(y, wn_ref[...])
    hn_ref[...] = hn
    rn_ref[...] = _dot(y, rw_ref[...]) + rb_ref[...]
    hsn_ref[...] = hn * dinv


def _stage_b(sp, h, res, dinv, dinv2, b, g, be, wn, rw, rb):
    fi = h.shape[1]
    fn = wn.shape[1]
    return pl.pallas_call(
        _kb_body,
        grid=(NB,),
        in_specs=[
            pl.BlockSpec((2, BN, fi), lambda i: (0, i, 0)),
            pl.BlockSpec((BN, fi), lambda i: (i, 0)),
            pl.BlockSpec((BN, fi), lambda i: (i, 0)),
            pl.BlockSpec((BN, 1), lambda i: (i, 0)),
            pl.BlockSpec((BN, 1), lambda i: (i, 0)),
            pl.BlockSpec((1, fi), lambda i: (0, 0)),
            pl.BlockSpec((1, fi), lambda i: (0, 0)),
            pl.BlockSpec((1, fi), lambda i: (0, 0)),
            pl.BlockSpec((fi, fn), lambda i: (0, 0)),
            pl.BlockSpec((fi, fn), lambda i: (0, 0)),
            pl.BlockSpec((1, fn), lambda i: (0, 0)),
        ],
        out_specs=[
            pl.BlockSpec((BN, fn), lambda i: (i, 0)),
            pl.BlockSpec((BN, fn), lambda i: (i, 0)),
            pl.BlockSpec((BN, fn), lambda i: (i, 0)),
        ],
        out_shape=[
            jax.ShapeDtypeStruct((N, fn), _f32),
            jax.ShapeDtypeStruct((N, fn), _f32),
            jax.ShapeDtypeStruct((N, fn), _f32),
        ],
    )(sp, h, res, dinv, dinv2, b.reshape(1, fi), g.reshape(1, fi),
      be.reshape(1, fi), wn, rw, rb.reshape(1, fn))


def _kl_body(sp_ref, h_ref, res_ref, dinv_ref, dinv2_ref, b_ref, g_ref,
             be_ref, x4_ref):
    t = (dinv_ref[...] * (sp_ref[0] + sp_ref[1])
         + h_ref[...] * dinv2_ref[...] + b_ref[...] + res_ref[...])
    m = jnp.mean(t, axis=1, keepdims=True)
    c = t - m
    v = jnp.mean(c * c, axis=1, keepdims=True)
    x4_ref[...] = jnp.maximum(
        c * jax.lax.rsqrt(v + 1e-5) * g_ref[...] + be_ref[...], 0.0)


def _stage_last(sp, h, res, dinv, dinv2, b, g, be):
    fi = h.shape[1]
    return pl.pallas_call(
        _kl_body,
        grid=(NB,),
        in_specs=[
            pl.BlockSpec((2, BN, fi), lambda i: (0, i, 0)),
            pl.BlockSpec((BN, fi), lambda i: (i, 0)),
            pl.BlockSpec((BN, fi), lambda i: (i, 0)),
            pl.BlockSpec((BN, 1), lambda i: (i, 0)),
            pl.BlockSpec((BN, 1), lambda i: (i, 0)),
            pl.BlockSpec((1, fi), lambda i: (0, 0)),
            pl.BlockSpec((1, fi), lambda i: (0, 0)),
            pl.BlockSpec((1, fi), lambda i: (0, 0)),
        ],
        out_specs=pl.BlockSpec((BN, fi), lambda i: (i, 0)),
        out_shape=jax.ShapeDtypeStruct((N, fi), _f32),
    )(sp, h, res, dinv, dinv2, b.reshape(1, fi), g.reshape(1, fi),
      be.reshape(1, fi))


def _kc_body(x4_ref, batch_ref, sums_ref, cnt_ref):
    i = pl.program_id(0)
    seg = jax.lax.broadcasted_iota(jnp.int32, (G, 1), 0)
    oh = jnp.where(batch_ref[0] == seg, 1.0, 0.0)
    psum = _dot(oh, x4_ref[...])
    pcnt = jnp.sum(oh, axis=1, keepdims=True)

    @pl.when(i == 0)
    def _():
        sums_ref[...] = psum
        cnt_ref[...] = pcnt

    @pl.when(i != 0)
    def _():
        sums_ref[...] += psum
        cnt_ref[...] += pcnt


def _stage_pool(x4, batch3):
    return pl.pallas_call(
        _kc_body,
        grid=(NB,),
        in_specs=[
            pl.BlockSpec((BN, OUT), lambda i: (i, 0)),
            pl.BlockSpec((1, 1, BN), lambda i: (i, 0, 0)),
        ],
        out_specs=[
            pl.BlockSpec((G, OUT), lambda i: (0, 0)),
            pl.BlockSpec((G, 1), lambda i: (0, 0)),
        ],
        out_shape=[
            jax.ShapeDtypeStruct((G, OUT), _f32),
            jax.ShapeDtypeStruct((G, 1), _f32),
        ],
    )(x4, batch3)


def _kd_body(sums_ref, cnt_ref, li0_ref, li1_ref, m1a_ref, m1b_ref,
             m1bias_ref, m2w_ref, m2b_ref, m3w_ref, m3b_ref, out_ref):
    emb = sums_ref[...] / jnp.maximum(cnt_ref[...], 1.0)
    ea = _dot(emb, m1a_ref[...])
    eb = _dot(emb, m1b_ref[...])
    gid = jax.lax.broadcasted_iota(jnp.int32, (1, G), 1)
    oh0 = jnp.where(li0_ref[0].reshape(BL, 1) == gid, 1.0, 0.0)
    oh1 = jnp.where(li1_ref[0].reshape(BL, 1) == gid, 1.0, 0.0)
    h = jnp.maximum(_dot(oh0, ea) + _dot(oh1, eb) + m1bias_ref[...], 0.0)
    h = jnp.maximum(_dot(h, m2w_ref[...]) + m2b_ref[...], 0.0)
    out_ref[...] = jax.nn.sigmoid(_dot(h, m3w_ref[...]) + m3b_ref[...])


def _stage_links(sums, cnt, li0_3, li1_3, m1w, m1b, m2w, m2b, m3w, m3b):
    return pl.pallas_call(
        _kd_body,
        grid=(NLB,),
        in_specs=[
            pl.BlockSpec((G, OUT), lambda i: (0, 0)),
            pl.BlockSpec((G, 1), lambda i: (0, 0)),
            pl.BlockSpec((1, 1, BL), lambda i: (i, 0, 0)),
            pl.BlockSpec((1, 1, BL), lambda i: (i, 0, 0)),
            pl.BlockSpec((OUT, OUT), lambda i: (0, 0)),
            pl.BlockSpec((OUT, OUT), lambda i: (0, 0)),
            pl.BlockSpec((1, OUT), lambda i: (0, 0)),
            pl.BlockSpec((OUT, OUT // 2), lambda i: (0, 0)),
            pl.BlockSpec((1, OUT // 2), lambda i: (0, 0)),
            pl.BlockSpec((OUT // 2, 1), lambda i: (0, 0)),
            pl.BlockSpec((1, 1), lambda i: (0, 0)),
        ],
        out_specs=pl.BlockSpec((BL, 1), lambda i: (i, 0)),
        out_shape=jax.ShapeDtypeStruct((L, 1), _f32),
    )(sums, cnt, li0_3, li1_3, m1w[:OUT], m1w[OUT:], m1b.reshape(1, OUT),
      m2w, m2b.reshape(1, OUT // 2), m3w, m3b.reshape(1, 1))


NPAD = 10112
CH = 80
EPAD = 2 * 16 * CH * 128
ROWS_PER_TILE = NPAD // 16
_OUT_CHUNKS = [(0, 128), (128, 128), (256, 128), (384, 128), (512, 120)]

_sc_mesh = plsc.VectorSubcoreMesh(core_axis_name="c", subcore_axis_name="s")


def _make_sc_scatter(f):
    @functools.partial(
        pl.kernel,
        mesh=_sc_mesh,
        compiler_params=pltpu.CompilerParams(use_tc_tiling_on_sc=False),
        out_type=jax.ShapeDtypeStruct((2, NPAD, f), jnp.float32),
        scratch_types=[
            pltpu.VMEM((CH, 128), jnp.int32),
            pltpu.VMEM((CH, 128), jnp.int32),
            pltpu.VMEM((128, f), jnp.float32),
            pltpu.VMEM_SHARED((NPAD, f), jnp.float32),
            pltpu.SemaphoreType.DMA,
        ],
    )
    def sc_scatter(hs_hbm, src_hbm, dst_hbm, zeros_hbm, out_hbm,
                   src_v, dst_v, rows_v, agg_sh, sem):
        c = lax.axis_index("c")
        s = lax.axis_index("s")
        r0 = s * ROWS_PER_TILE
        pltpu.sync_copy(zeros_hbm, rows_v)
        for k, sz in _OUT_CHUNKS:
            pltpu.sync_copy(rows_v.at[pl.ds(0, sz)],
                            agg_sh.at[pl.ds(r0 + k, sz)])
        pltpu.sync_copy(src_hbm.at[c, s], src_v)
        pltpu.sync_copy(dst_hbm.at[c, s], dst_v)
        plsc.subcore_barrier()

        def body(j, carry):
            pltpu.async_copy(hs_hbm.at[src_v.at[j]], rows_v, sem).wait()
            pltpu.sync_copy(rows_v, agg_sh.at[dst_v.at[j]], add=True)
            return carry

        lax.fori_loop(0, CH, body, 0)
        plsc.subcore_barrier()
        for k, sz in _OUT_CHUNKS:
            pltpu.sync_copy(agg_sh.at[pl.ds(r0 + k, sz)],
                            rows_v.at[pl.ds(0, sz)])
            pltpu.sync_copy(rows_v.at[pl.ds(0, sz)],
                            out_hbm.at[c, pl.ds(r0 + k, sz)])

    return sc_scatter


_sc_scatter = {f: _make_sc_scatter(f) for f in (H, H2, H3)}


@functools.partial(
    pl.kernel,
    mesh=_sc_mesh,
    compiler_params=pltpu.CompilerParams(use_tc_tiling_on_sc=False),
    out_type=[jax.ShapeDtypeStruct((NPAD,), jnp.float32),
              jax.ShapeDtypeStruct((NPAD,), jnp.float32)],
    scratch_types=[
        pltpu.VMEM((CH, 128), jnp.int32),
        pltpu.VMEM((128,), jnp.float32),
        pltpu.VMEM((ROWS_PER_TILE,), jnp.float32),
        pltpu.VMEM_SHARED((NPAD,), jnp.float32),
    ],
)
def _sc_degree(dst_hbm, zeros_hbm, ones_hbm, out0_hbm, out1_hbm,
               dst_v, ones_v, zbuf_v, deg_sh):
    c = lax.axis_index("c")
    s = lax.axis_index("s")
    r0 = s * ROWS_PER_TILE
    pltpu.sync_copy(zeros_hbm, zbuf_v)
    pltpu.sync_copy(zbuf_v, deg_sh.at[pl.ds(r0, ROWS_PER_TILE)])
    pltpu.sync_copy(ones_hbm, ones_v)
    pltpu.sync_copy(dst_hbm.at[c, s], dst_v)
    plsc.subcore_barrier()

    def body(j, carry):
        pltpu.sync_copy(ones_v, deg_sh.at[dst_v.at[j]], add=True)
        return carry

    lax.fori_loop(0, CH, body, 0)
    plsc.subcore_barrier()
    pltpu.sync_copy(deg_sh.at[pl.ds(r0, ROWS_PER_TILE)], zbuf_v)

    @pl.when(c == 0)
    def _():
        pltpu.sync_copy(zbuf_v, out0_hbm.at[pl.ds(r0, ROWS_PER_TILE)])

    @pl.when(c == 1)
    def _():
        pltpu.sync_copy(zbuf_v, out1_hbm.at[pl.ds(r0, ROWS_PER_TILE)])


def kernel(x, edge_index, batch, link_indices, W1, b1, g1, be1, W2, b2, R2w,
           R2b, g2, be2, W3, b3, R3w, R3b, g3, be3, W4, b4, R4w, R4b, g4,
           be4, M1w, M1b, M2w, M2b, M3w, M3b):
    src = edge_index[0]
    dst = edge_index[1]

    npad_e = EPAD - src.shape[0]
    src4 = jnp.concatenate([src, jnp.zeros((npad_e,), jnp.int32)])
    src4 = src4.reshape(2, 16, CH, 128)
    dst4 = jnp.concatenate([dst, jnp.full((npad_e,), N, jnp.int32)])
    dst4 = dst4.reshape(2, 16, CH, 128)
    zeros_w = jnp.zeros((128, H), _f32)
    zeros_1 = jnp.zeros((ROWS_PER_TILE,), _f32)
    ones_128 = jnp.ones((128,), _f32)

    dg0, dg1 = _sc_degree(dst4, zeros_1, ones_128)
    degp = jnp.stack([dg0[:N], dg1[:N]]).reshape(2, N, 1)

    h1, h1s, dinv, dinv2 = _stage_a(x, W1, degp)

    s1 = _sc_scatter[H](h1s, src4, dst4, zeros_w)[:, :N]
    h2, r2, h2s = _stage_b(s1, h1, x, dinv, dinv2, b1, g1, be1, W2, R2w, R2b)

    s2 = _sc_scatter[H2](h2s, src4, dst4, zeros_w[:, :H2])[:, :N]
    h3, r3, h3s = _stage_b(s2, h2, r2, dinv, dinv2, b2, g2, be2, W3, R3w,
                           R3b)

    s3 = _sc_scatter[H3](h3s, src4, dst4, zeros_w[:, :H3])[:, :N]
    h4, r4, h4s = _stage_b(s3, h3, r3, dinv, dinv2, b3, g3, be3, W4, R4w,
                           R4b)

    s4 = _sc_scatter[H2](h4s, src4, dst4, zeros_w[:, :H2])[:, :N]
    x4 = _stage_last(s4, h4, r4, dinv, dinv2, b4, g4, be4)

    batch3 = batch.reshape(NB, 1, BN)
    sums, cnt = _stage_pool(x4, batch3)

    li0_3 = link_indices[0].reshape(NLB, 1, BL)
    li1_3 = link_indices[1].reshape(NLB, 1, BL)
    out = _stage_links(sums, cnt, li0_3, li1_3, M1w, M1b, M2w, M2b, M3w,
                       M3b)
    return out.reshape(L)


# speedup vs baseline: 10.4983x; 1.3104x over previous
import functools

import jax
import jax.numpy as jnp
from jax import lax
from jax.experimental import pallas as pl
from jax.experimental.pallas import tpu as pltpu
from jax.experimental.pallas import tpu_sc as plsc

N = 10000
D = 128
H = 128
H2 = 64
H3 = 32
OUT = 64
G = 1024
L = 4096

BN = 1000
NB = N // BN
BL = 1024
NLB = L // BL

_f32 = jnp.float32


def _dot(a, b):
    return jax.lax.dot_general(a, b, (((1,), (0,)), ((), ())),
                               preferred_element_type=_f32)


def _ka_body(x_ref, w1_ref, degp_ref, h1_ref, h1s_ref, dinv_ref, dinv2_ref):
    deg = degp_ref[0] + degp_ref[1] + 1.0
    dinv = jax.lax.rsqrt(deg)
    h1 = _dot(x_ref[...], w1_ref[...])
    h1_ref[...] = h1
    h1s_ref[...] = h1 * dinv
    dinv_ref[...] = dinv
    dinv2_ref[...] = dinv * dinv


def _stage_a(x, w1, degp):
    return pl.pallas_call(
        _ka_body,
        grid=(NB,),
        in_specs=[
            pl.BlockSpec((BN, D), lambda i: (i, 0)),
            pl.BlockSpec((D, H), lambda i: (0, 0)),
            pl.BlockSpec((2, BN, 1), lambda i: (0, i, 0)),
        ],
        out_specs=[
            pl.BlockSpec((BN, H), lambda i: (i, 0)),
            pl.BlockSpec((BN, H), lambda i: (i, 0)),
            pl.BlockSpec((BN, 1), lambda i: (i, 0)),
            pl.BlockSpec((BN, 1), lambda i: (i, 0)),
        ],
        out_shape=[
            jax.ShapeDtypeStruct((N, H), _f32),
            jax.ShapeDtypeStruct((N, H), _f32),
            jax.ShapeDtypeStruct((N, 1), _f32),
            jax.ShapeDtypeStruct((N, 1), _f32),
        ],
    )(x, w1, degp)


def _kb_body(sp_ref, h_ref, res_ref, dinv_ref, dinv2_ref, b_ref, g_ref,
             be_ref, wn_ref, rw_ref, rb_ref, hn_ref, rn_ref, hsn_ref):
    dinv = dinv_ref[...]
    t = (dinv * (sp_ref[0] + sp_ref[1]) + h_ref[...] * dinv2_ref[...]
         + b_ref[...] + res_ref[...])
    m = jnp.mean(t, axis=1, keepdims=True)
    c = t - m
    v = jnp.mean(c * c, axis=1, keepdims=True)
    y = jnp.maximum(c * jax.lax.rsqrt(v + 1e-5) * g_ref[...] + be_ref[...],
                    0.0)
    hn = _dot(y, wn_ref[...])
    hn_ref[...] = hn
    rn_ref[...] = _dot(y, rw_ref[...]) + rb_ref[...]
    hsn_ref[...] = hn * dinv


def _stage_b(sp, h, res, dinv, dinv2, b, g, be, wn, rw, rb):
    fi = h.shape[1]
    fn = wn.shape[1]
    return pl.pallas_call(
        _kb_body,
        grid=(NB,),
        in_specs=[
            pl.BlockSpec((2, BN, fi), lambda i: (0, i, 0)),
            pl.BlockSpec((BN, fi), lambda i: (i, 0)),
            pl.BlockSpec((BN, fi), lambda i: (i, 0)),
            pl.BlockSpec((BN, 1), lambda i: (i, 0)),
            pl.BlockSpec((BN, 1), lambda i: (i, 0)),
            pl.BlockSpec((1, fi), lambda i: (0, 0)),
            pl.BlockSpec((1, fi), lambda i: (0, 0)),
            pl.BlockSpec((1, fi), lambda i: (0, 0)),
            pl.BlockSpec((fi, fn), lambda i: (0, 0)),
            pl.BlockSpec((fi, fn), lambda i: (0, 0)),
            pl.BlockSpec((1, fn), lambda i: (0, 0)),
        ],
        out_specs=[
            pl.BlockSpec((BN, fn), lambda i: (i, 0)),
            pl.BlockSpec((BN, fn), lambda i: (i, 0)),
            pl.BlockSpec((BN, fn), lambda i: (i, 0)),
        ],
        out_shape=[
            jax.ShapeDtypeStruct((N, fn), _f32),
            jax.ShapeDtypeStruct((N, fn), _f32),
            jax.ShapeDtypeStruct((N, fn), _f32),
        ],
    )(sp, h, res, dinv, dinv2, b.reshape(1, fi), g.reshape(1, fi),
      be.reshape(1, fi), wn, rw, rb.reshape(1, fn))


def _kl_body(sp_ref, h_ref, res_ref, dinv_ref, dinv2_ref, b_ref, g_ref,
             be_ref, x4_ref):
    t = (dinv_ref[...] * (sp_ref[0] + sp_ref[1])
         + h_ref[...] * dinv2_ref[...] + b_ref[...] + res_ref[...])
    m = jnp.mean(t, axis=1, keepdims=True)
    c = t - m
    v = jnp.mean(c * c, axis=1, keepdims=True)
    x4_ref[...] = jnp.maximum(
        c * jax.lax.rsqrt(v + 1e-5) * g_ref[...] + be_ref[...], 0.0)


def _stage_last(sp, h, res, dinv, dinv2, b, g, be):
    fi = h.shape[1]
    return pl.pallas_call(
        _kl_body,
        grid=(NB,),
        in_specs=[
            pl.BlockSpec((2, BN, fi), lambda i: (0, i, 0)),
            pl.BlockSpec((BN, fi), lambda i: (i, 0)),
            pl.BlockSpec((BN, fi), lambda i: (i, 0)),
            pl.BlockSpec((BN, 1), lambda i: (i, 0)),
            pl.BlockSpec((BN, 1), lambda i: (i, 0)),
            pl.BlockSpec((1, fi), lambda i: (0, 0)),
            pl.BlockSpec((1, fi), lambda i: (0, 0)),
            pl.BlockSpec((1, fi), lambda i: (0, 0)),
        ],
        out_specs=pl.BlockSpec((BN, fi), lambda i: (i, 0)),
        out_shape=jax.ShapeDtypeStruct((N, fi), _f32),
    )(sp, h, res, dinv, dinv2, b.reshape(1, fi), g.reshape(1, fi),
      be.reshape(1, fi))


def _kc_body(x4_ref, batch_ref, sums_ref, cnt_ref):
    i = pl.program_id(0)
    seg = jax.lax.broadcasted_iota(jnp.int32, (G, 1), 0)
    oh = jnp.where(batch_ref[0] == seg, 1.0, 0.0)
    psum = _dot(oh, x4_ref[...])
    pcnt = jnp.sum(oh, axis=1, keepdims=True)

    @pl.when(i == 0)
    def _():
        sums_ref[...] = psum
        cnt_ref[...] = pcnt

    @pl.when(i != 0)
    def _():
        sums_ref[...] += psum
        cnt_ref[...] += pcnt


def _stage_pool(x4, batch3):
    return pl.pallas_call(
        _kc_body,
        grid=(NB,),
        in_specs=[
            pl.BlockSpec((BN, OUT), lambda i: (i, 0)),
            pl.BlockSpec((1, 1, BN), lambda i: (i, 0, 0)),
        ],
        out_specs=[
            pl.BlockSpec((G, OUT), lambda i: (0, 0)),
            pl.BlockSpec((G, 1), lambda i: (0, 0)),
        ],
        out_shape=[
            jax.ShapeDtypeStruct((G, OUT), _f32),
            jax.ShapeDtypeStruct((G, 1), _f32),
        ],
    )(x4, batch3)


def _kd_body(sums_ref, cnt_ref, li0_ref, li1_ref, m1a_ref, m1b_ref,
             m1bias_ref, m2w_ref, m2b_ref, m3w_ref, m3b_ref, out_ref):
    emb = sums_ref[...] / jnp.maximum(cnt_ref[...], 1.0)
    ea = _dot(emb, m1a_ref[...])
    eb = _dot(emb, m1b_ref[...])
    gid = jax.lax.broadcasted_iota(jnp.int32, (1, G), 1)
    oh0 = jnp.where(li0_ref[0].reshape(BL, 1) == gid, 1.0, 0.0)
    oh1 = jnp.where(li1_ref[0].reshape(BL, 1) == gid, 1.0, 0.0)
    h = jnp.maximum(_dot(oh0, ea) + _dot(oh1, eb) + m1bias_ref[...], 0.0)
    h = jnp.maximum(_dot(h, m2w_ref[...]) + m2b_ref[...], 0.0)
    out_ref[...] = jax.nn.sigmoid(_dot(h, m3w_ref[...]) + m3b_ref[...])


def _stage_links(sums, cnt, li0_3, li1_3, m1w, m1b, m2w, m2b, m3w, m3b):
    return pl.pallas_call(
        _kd_body,
        grid=(NLB,),
        in_specs=[
            pl.BlockSpec((G, OUT), lambda i: (0, 0)),
            pl.BlockSpec((G, 1), lambda i: (0, 0)),
            pl.BlockSpec((1, 1, BL), lambda i: (i, 0, 0)),
            pl.BlockSpec((1, 1, BL), lambda i: (i, 0, 0)),
            pl.BlockSpec((OUT, OUT), lambda i: (0, 0)),
            pl.BlockSpec((OUT, OUT), lambda i: (0, 0)),
            pl.BlockSpec((1, OUT), lambda i: (0, 0)),
            pl.BlockSpec((OUT, OUT // 2), lambda i: (0, 0)),
            pl.BlockSpec((1, OUT // 2), lambda i: (0, 0)),
            pl.BlockSpec((OUT // 2, 1), lambda i: (0, 0)),
            pl.BlockSpec((1, 1), lambda i: (0, 0)),
        ],
        out_specs=pl.BlockSpec((BL, 1), lambda i: (i, 0)),
        out_shape=jax.ShapeDtypeStruct((L, 1), _f32),
    )(sums, cnt, li0_3, li1_3, m1w[:OUT], m1w[OUT:], m1b.reshape(1, OUT),
      m2w, m2b.reshape(1, OUT // 2), m3w, m3b.reshape(1, 1))


NPAD = 10112
CH = 80
EPAD = 2 * 16 * CH * 128
ROWS_PER_TILE = NPAD // 16
_OUT_CHUNKS = [(0, 128), (128, 128), (256, 128), (384, 128), (512, 120)]

_sc_mesh = plsc.VectorSubcoreMesh(core_axis_name="c", subcore_axis_name="s")


def _row_chunks(chunk):
    out, k = [], 0
    while k < ROWS_PER_TILE:
        out.append((k, min(chunk, ROWS_PER_TILE - k)))
        k += chunk
    return out


def _make_sc_scatter(f, chunk, nbuf, ch0, ch1):
    ch = max(ch0, ch1)
    out_chunks = _row_chunks(chunk)

    @functools.partial(
        pl.kernel,
        mesh=_sc_mesh,
        compiler_params=pltpu.CompilerParams(use_tc_tiling_on_sc=False),
        out_type=jax.ShapeDtypeStruct((2, NPAD, f), jnp.float32),
        scratch_types=[
            pltpu.VMEM((ch, chunk), jnp.int32),
            pltpu.VMEM((ch, chunk), jnp.int32),
            [pltpu.VMEM((chunk, f), jnp.float32) for _ in range(nbuf)],
            pltpu.VMEM_SHARED((NPAD, f), jnp.float32),
            pltpu.SemaphoreType.DMA,
            pltpu.SemaphoreType.DMA,
        ],
    )
    def sc_scatter(hs_hbm, src_hbm, dst_hbm, zeros_hbm, out_hbm,
                   src_v, dst_v, rows, agg_sh, gsem, ssem):
        c = lax.axis_index("c")
        s = lax.axis_index("s")
        r0 = s * ROWS_PER_TILE
        pltpu.sync_copy(zeros_hbm, rows[0])
        for k, sz in out_chunks:
            pltpu.sync_copy(rows[0].at[pl.ds(0, sz)],
                            agg_sh.at[pl.ds(r0 + k, sz)])
        pltpu.sync_copy(src_hbm.at[c, s], src_v)
        pltpu.sync_copy(dst_hbm.at[c, s], dst_v)
        plsc.subcore_barrier()

        my_ch = lax.select(c == 0, jnp.int32(ch0), jnp.int32(ch1))
        for b in range(nbuf):
            pltpu.async_copy(hs_hbm.at[src_v.at[b]], rows[b], gsem)

        def body(g, carry):
            for b in range(nbuf):
                j = nbuf * g + b
                pltpu.make_async_copy(hs_hbm.at[src_v.at[j]], rows[b],
                                      gsem).wait()
                sc = pltpu.async_copy(rows[b], agg_sh.at[dst_v.at[j]],
                                      ssem, add=True)
                sc.wait()

                @pl.when(j < my_ch - nbuf)
                def _():
                    pltpu.async_copy(hs_hbm.at[src_v.at[j + nbuf]],
                                     rows[b], gsem)
            return carry

        lax.fori_loop(0, my_ch // nbuf, body, 0)
        plsc.subcore_barrier()
        for k, sz in out_chunks:
            pltpu.sync_copy(agg_sh.at[pl.ds(r0 + k, sz)],
                            rows[0].at[pl.ds(0, sz)])
            pltpu.sync_copy(rows[0].at[pl.ds(0, sz)],
                            out_hbm.at[c, pl.ds(r0 + k, sz)])

    return sc_scatter


E_TOTAL = 320000
P0 = 0.50


def _split(chunk, nbuf):
    u = 16 * chunk
    ch0 = max(nbuf, int(round(P0 * E_TOTAL / u / nbuf)) * nbuf)
    rem = E_TOTAL - ch0 * u
    ch1 = max(nbuf, -(-rem // (u * nbuf)) * nbuf)
    return ch0, ch1


_SPLIT64 = _split(112, 2)
_SPLIT128 = _split(128, 4)

_sc_scatter = {H: _make_sc_scatter(H, 112, 2, *_SPLIT64),
               H2: _make_sc_scatter(H2, 128, 4, *_SPLIT128),
               H3: _make_sc_scatter(H3, 128, 4, *_SPLIT128)}


def _build_idx(vals, pad_val, chunk, ch0, ch1):
    chm = max(ch0, ch1)
    u = 16 * chunk
    a0 = vals[:ch0 * u].reshape(16, ch0, chunk)
    a0 = jnp.pad(a0, ((0, 0), (0, chm - ch0), (0, 0)),
                 constant_values=pad_val)
    rem = vals[ch0 * u:]
    a1 = jnp.concatenate(
        [rem, jnp.full((ch1 * u - rem.shape[0],), pad_val, jnp.int32)])
    a1 = jnp.pad(a1.reshape(16, ch1, chunk),
                 ((0, 0), (0, chm - ch1), (0, 0)),
                 constant_values=pad_val)
    return jnp.stack([a0, a1])


@functools.partial(
    pl.kernel,
    mesh=_sc_mesh,
    compiler_params=pltpu.CompilerParams(use_tc_tiling_on_sc=False),
    out_type=[jax.ShapeDtypeStruct((NPAD,), jnp.float32),
              jax.ShapeDtypeStruct((NPAD,), jnp.float32)],
    scratch_types=[
        pltpu.VMEM((CH, 128), jnp.int32),
        pltpu.VMEM((128,), jnp.float32),
        pltpu.VMEM((ROWS_PER_TILE,), jnp.float32),
        pltpu.VMEM_SHARED((NPAD,), jnp.float32),
    ],
)
def _sc_degree(dst_hbm, zeros_hbm, ones_hbm, out0_hbm, out1_hbm,
               dst_v, ones_v, zbuf_v, deg_sh):
    c = lax.axis_index("c")
    s = lax.axis_index("s")
    r0 = s * ROWS_PER_TILE
    pltpu.sync_copy(zeros_hbm, zbuf_v)
    pltpu.sync_copy(zbuf_v, deg_sh.at[pl.ds(r0, ROWS_PER_TILE)])
    pltpu.sync_copy(ones_hbm, ones_v)
    pltpu.sync_copy(dst_hbm.at[c, s], dst_v)
    plsc.subcore_barrier()

    def body(j, carry):
        pltpu.sync_copy(ones_v, deg_sh.at[dst_v.at[j]], add=True)
        return carry

    lax.fori_loop(0, CH, body, 0)
    plsc.subcore_barrier()
    pltpu.sync_copy(deg_sh.at[pl.ds(r0, ROWS_PER_TILE)], zbuf_v)

    @pl.when(c == 0)
    def _():
        pltpu.sync_copy(zbuf_v, out0_hbm.at[pl.ds(r0, ROWS_PER_TILE)])

    @pl.when(c == 1)
    def _():
        pltpu.sync_copy(zbuf_v, out1_hbm.at[pl.ds(r0, ROWS_PER_TILE)])


def kernel(x, edge_index, batch, link_indices, W1, b1, g1, be1, W2, b2, R2w,
           R2b, g2, be2, W3, b3, R3w, R3b, g3, be3, W4, b4, R4w, R4b, g4,
           be4, M1w, M1b, M2w, M2b, M3w, M3b):
    src = edge_index[0]
    dst = edge_index[1]

    npad_e = EPAD - src.shape[0]
    dst_p = jnp.concatenate([dst, jnp.full((npad_e,), N, jnp.int32)])
    dst4 = dst_p.reshape(2, 16, CH, 128)
    src4a = _build_idx(src, 0, 112, *_SPLIT64)
    dst4a = _build_idx(dst, N, 112, *_SPLIT64)
    src4b = _build_idx(src, 0, 128, *_SPLIT128)
    dst4b = _build_idx(dst, N, 128, *_SPLIT128)
    zeros_w = jnp.zeros((128, H), _f32)
    zeros_1 = jnp.zeros((ROWS_PER_TILE,), _f32)
    ones_128 = jnp.ones((128,), _f32)

    dg0, dg1 = _sc_degree(dst4, zeros_1, ones_128)
    degp = jnp.stack([dg0[:N], dg1[:N]]).reshape(2, N, 1)

    h1, h1s, dinv, dinv2 = _stage_a(x, W1, degp)

    s1 = _sc_scatter[H](h1s, src4a, dst4a, zeros_w[:112])[:, :N]
    h2, r2, h2s = _stage_b(s1, h1, x, dinv, dinv2, b1, g1, be1, W2, R2w, R2b)

    s2 = _sc_scatter[H2](h2s, src4b, dst4b, zeros_w[:, :H2])[:, :N]
    h3, r3, h3s = _stage_b(s2, h2, r2, dinv, dinv2, b2, g2, be2, W3, R3w,
                           R3b)

    s3 = _sc_scatter[H3](h3s, src4b, dst4b, zeros_w[:, :H3])[:, :N]
    h4, r4, h4s = _stage_b(s3, h3, r3, dinv, dinv2, b3, g3, be3, W4, R4w,
                           R4b)

    s4 = _sc_scatter[H2](h4s, src4b, dst4b, zeros_w[:, :H2])[:, :N]
    x4 = _stage_last(s4, h4, r4, dinv, dinv2, b4, g4, be4)

    batch3 = batch.reshape(NB, 1, BN)
    sums, cnt = _stage_pool(x4, batch3)

    li0_3 = link_indices[0].reshape(NLB, 1, BL)
    li1_3 = link_indices[1].reshape(NLB, 1, BL)
    out = _stage_links(sums, cnt, li0_3, li1_3, M1w, M1b, M2w, M2b, M3w,
                       M3b)
    return out.reshape(L)


# speedup vs baseline: 10.5036x; 1.0005x over previous
import functools

import jax
import jax.numpy as jnp
from jax import lax
from jax.experimental import pallas as pl
from jax.experimental.pallas import tpu as pltpu
from jax.experimental.pallas import tpu_sc as plsc

N = 10000
D = 128
H = 128
H2 = 64
H3 = 32
OUT = 64
G = 1024
L = 4096

BN = 1000
NB = N // BN
BL = 1024
NLB = L // BL

_f32 = jnp.float32


def _dot(a, b):
    return jax.lax.dot_general(a, b, (((1,), (0,)), ((), ())),
                               preferred_element_type=_f32)


def _ka_body(x_ref, w1_ref, degp_ref, h1_ref, h1s_ref, dinv_ref, dinv2_ref):
    deg = degp_ref[0] + degp_ref[1] + 1.0
    dinv = jax.lax.rsqrt(deg)
    h1 = _dot(x_ref[...], w1_ref[...])
    h1_ref[...] = h1
    h1s_ref[...] = h1 * dinv
    dinv_ref[...] = dinv
    dinv2_ref[...] = dinv * dinv


def _stage_a(x, w1, degp):
    return pl.pallas_call(
        _ka_body,
        grid=(NB,),
        in_specs=[
            pl.BlockSpec((BN, D), lambda i: (i, 0)),
            pl.BlockSpec((D, H), lambda i: (0, 0)),
            pl.BlockSpec((2, BN, 1), lambda i: (0, i, 0)),
        ],
        out_specs=[
            pl.BlockSpec((BN, H), lambda i: (i, 0)),
            pl.BlockSpec((BN, H), lambda i: (i, 0)),
            pl.BlockSpec((BN, 1), lambda i: (i, 0)),
            pl.BlockSpec((BN, 1), lambda i: (i, 0)),
        ],
        out_shape=[
            jax.ShapeDtypeStruct((N, H), _f32),
            jax.ShapeDtypeStruct((N, H), _f32),
            jax.ShapeDtypeStruct((N, 1), _f32),
            jax.ShapeDtypeStruct((N, 1), _f32),
        ],
    )(x, w1, degp)


def _kb_body(sp_ref, h_ref, res_ref, dinv_ref, dinv2_ref, b_ref, g_ref,
             be_ref, wn_ref, rw_ref, rb_ref, hn_ref, rn_ref, hsn_ref):
    dinv = dinv_ref[...]
    t = (dinv * (sp_ref[0] + sp_ref[1]) + h_ref[...] * dinv2_ref[...]
         + b_ref[...] + res_ref[...])
    m = jnp.mean(t, axis=1, keepdims=True)
    c = t - m
    v = jnp.mean(c * c, axis=1, keepdims=True)
    y = jnp.maximum(c * jax.lax.rsqrt(v + 1e-5) * g_ref[...] + be_ref[...],
                    0.0)
    hn = _dot(y, wn_ref[...])
    hn_ref[...] = hn
    rn_ref[...] = _dot(y, rw_ref[...]) + rb_ref[...]
    hsn_ref[...] = hn * dinv


def _stage_b(sp, h, res, dinv, dinv2, b, g, be, wn, rw, rb):
    fi = h.shape[1]
    fn = wn.shape[1]
    return pl.pallas_call(
        _kb_body,
        grid=(NB,),
        in_specs=[
            pl.BlockSpec((2, BN, fi), lambda i: (0, i, 0)),
            pl.BlockSpec((BN, fi), lambda i: (i, 0)),
            pl.BlockSpec((BN, fi), lambda i: (i, 0)),
            pl.BlockSpec((BN, 1), lambda i: (i, 0)),
            pl.BlockSpec((BN, 1), lambda i: (i, 0)),
            pl.BlockSpec((1, fi), lambda i: (0, 0)),
            pl.BlockSpec((1, fi), lambda i: (0, 0)),
            pl.BlockSpec((1, fi), lambda i: (0, 0)),
            pl.BlockSpec((fi, fn), lambda i: (0, 0)),
            pl.BlockSpec((fi, fn), lambda i: (0, 0)),
            pl.BlockSpec((1, fn), lambda i: (0, 0)),
        ],
        out_specs=[
            pl.BlockSpec((BN, fn), lambda i: (i, 0)),
            pl.BlockSpec((BN, fn), lambda i: (i, 0)),
            pl.BlockSpec((BN, fn), lambda i: (i, 0)),
        ],
        out_shape=[
            jax.ShapeDtypeStruct((N, fn), _f32),
            jax.ShapeDtypeStruct((N, fn), _f32),
            jax.ShapeDtypeStruct((N, fn), _f32),
        ],
    )(sp, h, res, dinv, dinv2, b.reshape(1, fi), g.reshape(1, fi),
      be.reshape(1, fi), wn, rw, rb.reshape(1, fn))


def _kl_body(sp_ref, h_ref, res_ref, dinv_ref, dinv2_ref, b_ref, g_ref,
             be_ref, x4_ref):
    t = (dinv_ref[...] * (sp_ref[0] + sp_ref[1])
         + h_ref[...] * dinv2_ref[...] + b_ref[...] + res_ref[...])
    m = jnp.mean(t, axis=1, keepdims=True)
    c = t - m
    v = jnp.mean(c * c, axis=1, keepdims=True)
    x4_ref[...] = jnp.maximum(
        c * jax.lax.rsqrt(v + 1e-5) * g_ref[...] + be_ref[...], 0.0)


def _stage_last(sp, h, res, dinv, dinv2, b, g, be):
    fi = h.shape[1]
    return pl.pallas_call(
        _kl_body,
        grid=(NB,),
        in_specs=[
            pl.BlockSpec((2, BN, fi), lambda i: (0, i, 0)),
            pl.BlockSpec((BN, fi), lambda i: (i, 0)),
            pl.BlockSpec((BN, fi), lambda i: (i, 0)),
            pl.BlockSpec((BN, 1), lambda i: (i, 0)),
            pl.BlockSpec((BN, 1), lambda i: (i, 0)),
            pl.BlockSpec((1, fi), lambda i: (0, 0)),
            pl.BlockSpec((1, fi), lambda i: (0, 0)),
            pl.BlockSpec((1, fi), lambda i: (0, 0)),
        ],
        out_specs=pl.BlockSpec((BN, fi), lambda i: (i, 0)),
        out_shape=jax.ShapeDtypeStruct((N, fi), _f32),
    )(sp, h, res, dinv, dinv2, b.reshape(1, fi), g.reshape(1, fi),
      be.reshape(1, fi))


def _kc_body(x4_ref, batch_ref, sums_ref, cnt_ref):
    i = pl.program_id(0)
    seg = jax.lax.broadcasted_iota(jnp.int32, (G, 1), 0)
    oh = jnp.where(batch_ref[0] == seg, 1.0, 0.0)
    psum = _dot(oh, x4_ref[...])
    pcnt = jnp.sum(oh, axis=1, keepdims=True)

    @pl.when(i == 0)
    def _():
        sums_ref[...] = psum
        cnt_ref[...] = pcnt

    @pl.when(i != 0)
    def _():
        sums_ref[...] += psum
        cnt_ref[...] += pcnt


def _stage_pool(x4, batch3):
    return pl.pallas_call(
        _kc_body,
        grid=(NB,),
        in_specs=[
            pl.BlockSpec((BN, OUT), lambda i: (i, 0)),
            pl.BlockSpec((1, 1, BN), lambda i: (i, 0, 0)),
        ],
        out_specs=[
            pl.BlockSpec((G, OUT), lambda i: (0, 0)),
            pl.BlockSpec((G, 1), lambda i: (0, 0)),
        ],
        out_shape=[
            jax.ShapeDtypeStruct((G, OUT), _f32),
            jax.ShapeDtypeStruct((G, 1), _f32),
        ],
    )(x4, batch3)


def _kd_body(sums_ref, cnt_ref, li0_ref, li1_ref, m1a_ref, m1b_ref,
             m1bias_ref, m2w_ref, m2b_ref, m3w_ref, m3b_ref, out_ref):
    emb = sums_ref[...] / jnp.maximum(cnt_ref[...], 1.0)
    ea = _dot(emb, m1a_ref[...])
    eb = _dot(emb, m1b_ref[...])
    gid = jax.lax.broadcasted_iota(jnp.int32, (1, G), 1)
    oh0 = jnp.where(li0_ref[0].reshape(BL, 1) == gid, 1.0, 0.0)
    oh1 = jnp.where(li1_ref[0].reshape(BL, 1) == gid, 1.0, 0.0)
    h = jnp.maximum(_dot(oh0, ea) + _dot(oh1, eb) + m1bias_ref[...], 0.0)
    h = jnp.maximum(_dot(h, m2w_ref[...]) + m2b_ref[...], 0.0)
    out_ref[...] = jax.nn.sigmoid(_dot(h, m3w_ref[...]) + m3b_ref[...])


def _stage_links(sums, cnt, li0_3, li1_3, m1w, m1b, m2w, m2b, m3w, m3b):
    return pl.pallas_call(
        _kd_body,
        grid=(NLB,),
        in_specs=[
            pl.BlockSpec((G, OUT), lambda i: (0, 0)),
            pl.BlockSpec((G, 1), lambda i: (0, 0)),
            pl.BlockSpec((1, 1, BL), lambda i: (i, 0, 0)),
            pl.BlockSpec((1, 1, BL), lambda i: (i, 0, 0)),
            pl.BlockSpec((OUT, OUT), lambda i: (0, 0)),
            pl.BlockSpec((OUT, OUT), lambda i: (0, 0)),
            pl.BlockSpec((1, OUT), lambda i: (0, 0)),
            pl.BlockSpec((OUT, OUT // 2), lambda i: (0, 0)),
            pl.BlockSpec((1, OUT // 2), lambda i: (0, 0)),
            pl.BlockSpec((OUT // 2, 1), lambda i: (0, 0)),
            pl.BlockSpec((1, 1), lambda i: (0, 0)),
        ],
        out_specs=pl.BlockSpec((BL, 1), lambda i: (i, 0)),
        out_shape=jax.ShapeDtypeStruct((L, 1), _f32),
    )(sums, cnt, li0_3, li1_3, m1w[:OUT], m1w[OUT:], m1b.reshape(1, OUT),
      m2w, m2b.reshape(1, OUT // 2), m3w, m3b.reshape(1, 1))


NPAD = 10112
CH = 80
EPAD = 2 * 16 * CH * 128
ROWS_PER_TILE = NPAD // 16
_OUT_CHUNKS = [(0, 128), (128, 128), (256, 128), (384, 128), (512, 120)]

_sc_mesh = plsc.VectorSubcoreMesh(core_axis_name="c", subcore_axis_name="s")


def _row_chunks(chunk):
    out, k = [], 0
    while k < ROWS_PER_TILE:
        out.append((k, min(chunk, ROWS_PER_TILE - k)))
        k += chunk
    return out


def _make_sc_scatter(f, chunk, nbuf, ch0, ch1):
    ch = max(ch0, ch1)
    out_chunks = _row_chunks(chunk)

    @functools.partial(
        pl.kernel,
        mesh=_sc_mesh,
        compiler_params=pltpu.CompilerParams(use_tc_tiling_on_sc=False),
        out_type=jax.ShapeDtypeStruct((2, NPAD, f), jnp.float32),
        scratch_types=[
            pltpu.VMEM((ch, chunk), jnp.int32),
            pltpu.VMEM((ch, chunk), jnp.int32),
            [pltpu.VMEM((chunk, f), jnp.float32) for _ in range(nbuf)],
            pltpu.VMEM_SHARED((NPAD, f), jnp.float32),
            pltpu.SemaphoreType.DMA,
            pltpu.SemaphoreType.DMA,
        ],
    )
    def sc_scatter(hs_hbm, src_hbm, dst_hbm, zeros_hbm, out_hbm,
                   src_v, dst_v, rows, agg_sh, gsem, ssem):
        c = lax.axis_index("c")
        s = lax.axis_index("s")
        r0 = s * ROWS_PER_TILE
        pltpu.sync_copy(zeros_hbm, rows[0])
        for k, sz in out_chunks:
            pltpu.sync_copy(rows[0].at[pl.ds(0, sz)],
                            agg_sh.at[pl.ds(r0 + k, sz)])
        pltpu.sync_copy(src_hbm.at[c, s], src_v)
        pltpu.sync_copy(dst_hbm.at[c, s], dst_v)
        plsc.subcore_barrier()

        my_ch = lax.select(c == 0, jnp.int32(ch0), jnp.int32(ch1))
        for b in range(nbuf):
            pltpu.async_copy(hs_hbm.at[src_v.at[b]], rows[b], gsem)

        def body(g, carry):
            for b in range(nbuf):
                j = nbuf * g + b
                pltpu.make_async_copy(hs_hbm.at[src_v.at[j]], rows[b],
                                      gsem).wait()
                sc = pltpu.async_copy(rows[b], agg_sh.at[dst_v.at[j]],
                                      ssem, add=True)
                sc.wait()

                @pl.when(j < my_ch - nbuf)
                def _():
                    pltpu.async_copy(hs_hbm.at[src_v.at[j + nbuf]],
                                     rows[b], gsem)
            return carry

        lax.fori_loop(0, my_ch // nbuf, body, 0)
        plsc.subcore_barrier()
        for k, sz in out_chunks:
            pltpu.sync_copy(agg_sh.at[pl.ds(r0 + k, sz)],
                            rows[0].at[pl.ds(0, sz)])
            pltpu.sync_copy(rows[0].at[pl.ds(0, sz)],
                            out_hbm.at[c, pl.ds(r0 + k, sz)])

    return sc_scatter


E_TOTAL = 320000
P0 = 0.50


def _split(chunk, nbuf):
    u = 16 * chunk
    ch0 = max(nbuf, int(round(P0 * E_TOTAL / u / nbuf)) * nbuf)
    rem = E_TOTAL - ch0 * u
    ch1 = max(nbuf, -(-rem // (u * nbuf)) * nbuf)
    return ch0, ch1


_SPLIT64 = _split(112, 2)
_SPLIT128 = _split(128, 5)

_sc_scatter = {H: _make_sc_scatter(H, 112, 2, *_SPLIT64),
               H2: _make_sc_scatter(H2, 128, 5, *_SPLIT128),
               H3: _make_sc_scatter(H3, 128, 5, *_SPLIT128)}


def _build_idx(vals, pad_val, chunk, ch0, ch1):
    chm = max(ch0, ch1)
    u = 16 * chunk
    a0 = vals[:ch0 * u].reshape(16, ch0, chunk)
    a0 = jnp.pad(a0, ((0, 0), (0, chm - ch0), (0, 0)),
                 constant_values=pad_val)
    rem = vals[ch0 * u:]
    a1 = jnp.concatenate(
        [rem, jnp.full((ch1 * u - rem.shape[0],), pad_val, jnp.int32)])
    a1 = jnp.pad(a1.reshape(16, ch1, chunk),
                 ((0, 0), (0, chm - ch1), (0, 0)),
                 constant_values=pad_val)
    return jnp.stack([a0, a1])


@functools.partial(
    pl.kernel,
    mesh=_sc_mesh,
    compiler_params=pltpu.CompilerParams(use_tc_tiling_on_sc=False),
    out_type=[jax.ShapeDtypeStruct((NPAD,), jnp.float32),
              jax.ShapeDtypeStruct((NPAD,), jnp.float32)],
    scratch_types=[
        pltpu.VMEM((CH, 128), jnp.int32),
        pltpu.VMEM((128,), jnp.float32),
        pltpu.VMEM((ROWS_PER_TILE,), jnp.float32),
        pltpu.VMEM_SHARED((NPAD,), jnp.float32),
    ],
)
def _sc_degree(dst_hbm, zeros_hbm, ones_hbm, out0_hbm, out1_hbm,
               dst_v, ones_v, zbuf_v, deg_sh):
    c = lax.axis_index("c")
    s = lax.axis_index("s")
    r0 = s * ROWS_PER_TILE
    pltpu.sync_copy(zeros_hbm, zbuf_v)
    pltpu.sync_copy(zbuf_v, deg_sh.at[pl.ds(r0, ROWS_PER_TILE)])
    pltpu.sync_copy(ones_hbm, ones_v)
    pltpu.sync_copy(dst_hbm.at[c, s], dst_v)
    plsc.subcore_barrier()

    def body(j, carry):
        pltpu.sync_copy(ones_v, deg_sh.at[dst_v.at[j]], add=True)
        return carry

    lax.fori_loop(0, CH, body, 0)
    plsc.subcore_barrier()
    pltpu.sync_copy(deg_sh.at[pl.ds(r0, ROWS_PER_TILE)], zbuf_v)

    @pl.when(c == 0)
    def _():
        pltpu.sync_copy(zbuf_v, out0_hbm.at[pl.ds(r0, ROWS_PER_TILE)])

    @pl.when(c == 1)
    def _():
        pltpu.sync_copy(zbuf_v, out1_hbm.at[pl.ds(r0, ROWS_PER_TILE)])


def kernel(x, edge_index, batch, link_indices, W1, b1, g1, be1, W2, b2, R2w,
           R2b, g2, be2, W3, b3, R3w, R3b, g3, be3, W4, b4, R4w, R4b, g4,
           be4, M1w, M1b, M2w, M2b, M3w, M3b):
    src = edge_index[0]
    dst = edge_index[1]

    npad_e = EPAD - src.shape[0]
    dst_p = jnp.concatenate([dst, jnp.full((npad_e,), N, jnp.int32)])
    dst4 = dst_p.reshape(2, 16, CH, 128)
    src4a = _build_idx(src, 0, 112, *_SPLIT64)
    dst4a = _build_idx(dst, N, 112, *_SPLIT64)
    src4b = _build_idx(src, 0, 128, *_SPLIT128)
    dst4b = _build_idx(dst, N, 128, *_SPLIT128)
    zeros_w = jnp.zeros((128, H), _f32)
    zeros_1 = jnp.zeros((ROWS_PER_TILE,), _f32)
    ones_128 = jnp.ones((128,), _f32)

    dg0, dg1 = _sc_degree(dst4, zeros_1, ones_128)
    degp = jnp.stack([dg0[:N], dg1[:N]]).reshape(2, N, 1)

    h1, h1s, dinv, dinv2 = _stage_a(x, W1, degp)

    s1 = _sc_scatter[H](h1s, src4a, dst4a, zeros_w[:112])[:, :N]
    h2, r2, h2s = _stage_b(s1, h1, x, dinv, dinv2, b1, g1, be1, W2, R2w, R2b)

    s2 = _sc_scatter[H2](h2s, src4b, dst4b, zeros_w[:, :H2])[:, :N]
    h3, r3, h3s = _stage_b(s2, h2, r2, dinv, dinv2, b2, g2, be2, W3, R3w,
                           R3b)

    s3 = _sc_scatter[H3](h3s, src4b, dst4b, zeros_w[:, :H3])[:, :N]
    h4, r4, h4s = _stage_b(s3, h3, r3, dinv, dinv2, b3, g3, be3, W4, R4w,
                           R4b)

    s4 = _sc_scatter[H2](h4s, src4b, dst4b, zeros_w[:, :H2])[:, :N]
    x4 = _stage_last(s4, h4, r4, dinv, dinv2, b4, g4, be4)

    batch3 = batch.reshape(NB, 1, BN)
    sums, cnt = _stage_pool(x4, batch3)

    li0_3 = link_indices[0].reshape(NLB, 1, BL)
    li1_3 = link_indices[1].reshape(NLB, 1, BL)
    out = _stage_links(sums, cnt, li0_3, li1_3, M1w, M1b, M2w, M2b, M3w,
                       M3b)
    return out.reshape(L)
